# baseline (speedup 1.0000x reference)
import jax
import jax.numpy as jnp
from jax.experimental import pallas as pl
from jax.experimental.pallas import tpu as pltpu

K = 64
P = 1024 * 1024
NS = 16
B = P // NS


def _one_hot(x_bf):
    one = jnp.bfloat16(1.0)
    zero = jnp.bfloat16(0.0)
    ids8 = jax.lax.broadcasted_iota(jnp.int32, (8, 1), 0)
    parts = []
    for g in range(8):
        ids = (ids8 + 8 * g).astype(jnp.bfloat16)
        parts.append(jnp.where(x_bf == ids, one, zero))
    return jnp.concatenate(parts, axis=0)


def _body(pm_ref, tm_ref, out_ref, acc_ref):
    s = pl.program_id(0)
    pm = jnp.broadcast_to(pm_ref[0], (8, B)).astype(jnp.bfloat16)
    tm = jnp.broadcast_to(tm_ref[0], (8, B)).astype(jnp.bfloat16)
    p_oh = _one_hot(pm)
    t_oh = _one_hot(tm)
    part = jax.lax.dot_general(
        p_oh, t_oh, (((1,), (1,)), ((), ())),
        preferred_element_type=jnp.float32)

    @pl.when(s == 0)
    def _init():
        acc_ref[...] = part

    @pl.when(s > 0)
    def _acc():
        acc_ref[...] += part

    @pl.when(s == NS - 1)
    def _finalize():
        inter = acc_ref[...]
        area_p = jnp.sum(inter, axis=1, keepdims=True)
        area_t = jnp.sum(inter, axis=0, keepdims=True)
        union = area_p + area_t - inter
        iou = jnp.where(union > 0, inter / jnp.maximum(union, 1.0), 0.0)

        col = jax.lax.broadcasted_iota(jnp.int32, (K, K), 1)
        row = jax.lax.broadcasted_iota(jnp.int32, (K, K), 0)
        iou_c = jnp.where(col == 0, 0.0, iou)
        iou_r = jnp.where(row == 0, 0.0, iou)

        max_p = jnp.max(iou_c, axis=1, keepdims=True)
        max_t = jnp.max(iou_r, axis=0, keepdims=True)

        rid = jax.lax.broadcasted_iota(jnp.int32, (K, 1), 0)
        cid = jax.lax.broadcasted_iota(jnp.int32, (1, K), 1)
        pres_p = (area_p > 0) & (rid > 0)
        pres_t = (area_t > 0) & (cid > 0)

        loss = (jnp.sum(jnp.where(pres_p, 1.0 - max_p, 0.0),
                        axis=0, keepdims=True)
                + jnp.sum(jnp.where(pres_t, 1.0 - max_t, 0.0),
                          axis=1, keepdims=True))
        n = (jnp.sum(pres_p.astype(jnp.float32), axis=0, keepdims=True)
             + jnp.sum(pres_t.astype(jnp.float32), axis=1, keepdims=True))
        out_ref[...] = jnp.where(n > 0, loss / jnp.maximum(n, 1.0), 0.0)


def kernel(pred_mask, true_mask):
    pm = pred_mask.reshape(NS, 1, B)
    tm = true_mask.reshape(NS, 1, B)

    loss = pl.pallas_call(
        _body,
        out_shape=jax.ShapeDtypeStruct((1, 1), jnp.float32),
        grid=(NS,),
        in_specs=[
            pl.BlockSpec((1, 1, B), lambda s: (s, 0, 0)),
            pl.BlockSpec((1, 1, B), lambda s: (s, 0, 0)),
        ],
        out_specs=pl.BlockSpec((1, 1), lambda s: (0, 0)),
        scratch_shapes=[pltpu.VMEM((K, K), jnp.float32)],
        name="iou_loss_fused",
    )(pm, tm)
    return loss[0, 0]


# speedup vs baseline: 1.2448x; 1.2448x over previous
import jax
import jax.numpy as jnp
from jax.experimental import pallas as pl
from jax.experimental.pallas import tpu as pltpu

K = 64
P = 1024 * 1024
NS = 32
B = P // NS


def _one_hot(x_bf):
    one = jnp.bfloat16(1.0)
    zero = jnp.bfloat16(0.0)
    ids = jax.lax.broadcasted_iota(jnp.int32, (K, 1), 0).astype(jnp.bfloat16)
    return jnp.where(x_bf == ids, one, zero)


def _body(pm_ref, tm_ref, out_ref, acc_ref):
    s = pl.program_id(0)
    p_oh = _one_hot(pm_ref[0].astype(jnp.bfloat16))
    t_oh = _one_hot(tm_ref[0].astype(jnp.bfloat16))
    part = jax.lax.dot_general(
        p_oh, t_oh, (((1,), (1,)), ((), ())),
        preferred_element_type=jnp.float32)

    @pl.when(s == 0)
    def _init():
        acc_ref[...] = part

    @pl.when(s > 0)
    def _acc():
        acc_ref[...] += part

    @pl.when(s == NS - 1)
    def _finalize():
        inter = acc_ref[...]
        area_p = jnp.sum(inter, axis=1, keepdims=True)
        area_t = jnp.sum(inter, axis=0, keepdims=True)
        union = area_p + area_t - inter
        iou = jnp.where(union > 0, inter / jnp.maximum(union, 1.0), 0.0)

        col = jax.lax.broadcasted_iota(jnp.int32, (K, K), 1)
        row = jax.lax.broadcasted_iota(jnp.int32, (K, K), 0)
        iou_c = jnp.where(col == 0, 0.0, iou)
        iou_r = jnp.where(row == 0, 0.0, iou)

        max_p = jnp.max(iou_c, axis=1, keepdims=True)
        max_t = jnp.max(iou_r, axis=0, keepdims=True)

        rid = jax.lax.broadcasted_iota(jnp.int32, (K, 1), 0)
        cid = jax.lax.broadcasted_iota(jnp.int32, (1, K), 1)
        pres_p = (area_p > 0) & (rid > 0)
        pres_t = (area_t > 0) & (cid > 0)

        loss = (jnp.sum(jnp.where(pres_p, 1.0 - max_p, 0.0),
                        axis=0, keepdims=True)
                + jnp.sum(jnp.where(pres_t, 1.0 - max_t, 0.0),
                          axis=1, keepdims=True))
        n = (jnp.sum(pres_p.astype(jnp.float32), axis=0, keepdims=True)
             + jnp.sum(pres_t.astype(jnp.float32), axis=1, keepdims=True))
        out_ref[...] = jnp.where(n > 0, loss / jnp.maximum(n, 1.0), 0.0)


def kernel(pred_mask, true_mask):
    pm = pred_mask.reshape(NS, 1, B)
    tm = true_mask.reshape(NS, 1, B)

    loss = pl.pallas_call(
        _body,
        out_shape=jax.ShapeDtypeStruct((1, 1), jnp.float32),
        grid=(NS,),
        in_specs=[
            pl.BlockSpec((1, 1, B), lambda s: (s, 0, 0)),
            pl.BlockSpec((1, 1, B), lambda s: (s, 0, 0)),
        ],
        out_specs=pl.BlockSpec((1, 1), lambda s: (0, 0)),
        scratch_shapes=[pltpu.VMEM((K, K), jnp.float32)],
        name="iou_loss_fused",
    )(pm, tm)
    return loss[0, 0]


# speedup vs baseline: 1.9101x; 1.5344x over previous
import jax
import jax.numpy as jnp
from jax.experimental import pallas as pl
from jax.experimental.pallas import tpu as pltpu

K = 64
H = 1024
W = 1024
R = 32
NS = H // R


def _body(pm_ref, tm_ref, out_ref, acc_ref):
    s = pl.program_id(0)
    pm = pm_ref[0].astype(jnp.bfloat16)
    tm = tm_ref[0].astype(jnp.bfloat16)
    ids = jax.lax.broadcasted_iota(jnp.int32, (K, 1), 0).astype(jnp.bfloat16)
    one = jnp.bfloat16(1.0)
    zero = jnp.bfloat16(0.0)

    part = jnp.zeros((K, K), jnp.float32)
    for r in range(R):
        p_oh = jnp.where(pm[r:r + 1, :] == ids, one, zero)
        t_oh = jnp.where(tm[r:r + 1, :] == ids, one, zero)
        part += jax.lax.dot_general(
            p_oh, t_oh, (((1,), (1,)), ((), ())),
            preferred_element_type=jnp.float32)

    @pl.when(s == 0)
    def _init():
        acc_ref[...] = part

    @pl.when(s > 0)
    def _acc():
        acc_ref[...] += part

    @pl.when(s == NS - 1)
    def _finalize():
        inter = acc_ref[...]
        area_p = jnp.sum(inter, axis=1, keepdims=True)
        area_t = jnp.sum(inter, axis=0, keepdims=True)
        union = area_p + area_t - inter
        iou = jnp.where(union > 0, inter / jnp.maximum(union, 1.0), 0.0)

        col = jax.lax.broadcasted_iota(jnp.int32, (K, K), 1)
        row = jax.lax.broadcasted_iota(jnp.int32, (K, K), 0)
        iou_c = jnp.where(col == 0, 0.0, iou)
        iou_r = jnp.where(row == 0, 0.0, iou)

        max_p = jnp.max(iou_c, axis=1, keepdims=True)
        max_t = jnp.max(iou_r, axis=0, keepdims=True)

        rid = jax.lax.broadcasted_iota(jnp.int32, (K, 1), 0)
        cid = jax.lax.broadcasted_iota(jnp.int32, (1, K), 1)
        pres_p = (area_p > 0) & (rid > 0)
        pres_t = (area_t > 0) & (cid > 0)

        loss = (jnp.sum(jnp.where(pres_p, 1.0 - max_p, 0.0),
                        axis=0, keepdims=True)
                + jnp.sum(jnp.where(pres_t, 1.0 - max_t, 0.0),
                          axis=1, keepdims=True))
        n = (jnp.sum(pres_p.astype(jnp.float32), axis=0, keepdims=True)
             + jnp.sum(pres_t.astype(jnp.float32), axis=1, keepdims=True))
        out_ref[...] = jnp.where(n > 0, loss / jnp.maximum(n, 1.0), 0.0)


def kernel(pred_mask, true_mask):
    loss = pl.pallas_call(
        _body,
        out_shape=jax.ShapeDtypeStruct((1, 1), jnp.float32),
        grid=(NS,),
        in_specs=[
            pl.BlockSpec((1, R, W), lambda s: (0, s, 0)),
            pl.BlockSpec((1, R, W), lambda s: (0, s, 0)),
        ],
        out_specs=pl.BlockSpec((1, 1), lambda s: (0, 0)),
        scratch_shapes=[pltpu.VMEM((K, K), jnp.float32)],
        name="iou_loss_fused",
    )(pred_mask, true_mask)
    return loss[0, 0]


# speedup vs baseline: 1.9243x; 1.0075x over previous
import jax
import jax.numpy as jnp
from jax.experimental import pallas as pl
from jax.experimental.pallas import tpu as pltpu

K = 64
H = 1024
W = 1024
R = 32
NS = H // R


def _body(pm_ref, tm_ref, out_ref, acc_ref):
    s = pl.program_id(0)
    pm = pm_ref[0].astype(jnp.bfloat16)
    tm = tm_ref[0].astype(jnp.bfloat16)
    ids = jax.lax.broadcasted_iota(jnp.int32, (K, 1), 0).astype(jnp.bfloat16)
    one = jnp.bfloat16(1.0)
    zero = jnp.bfloat16(0.0)

    p_oh = jnp.concatenate(
        [jnp.where(pm[r:r + 1, :] == ids, one, zero) for r in range(R)],
        axis=1)
    t_oh = jnp.concatenate(
        [jnp.where(tm[r:r + 1, :] == ids, one, zero) for r in range(R)],
        axis=1)
    part = jax.lax.dot_general(
        p_oh, t_oh, (((1,), (1,)), ((), ())),
        preferred_element_type=jnp.float32)

    @pl.when(s == 0)
    def _init():
        acc_ref[...] = part

    @pl.when(s > 0)
    def _acc():
        acc_ref[...] += part

    @pl.when(s == NS - 1)
    def _finalize():
        inter = acc_ref[...]
        area_p = jnp.sum(inter, axis=1, keepdims=True)
        area_t = jnp.sum(inter, axis=0, keepdims=True)
        union = area_p + area_t - inter
        iou = jnp.where(union > 0, inter / jnp.maximum(union, 1.0), 0.0)

        col = jax.lax.broadcasted_iota(jnp.int32, (K, K), 1)
        row = jax.lax.broadcasted_iota(jnp.int32, (K, K), 0)
        iou_c = jnp.where(col == 0, 0.0, iou)
        iou_r = jnp.where(row == 0, 0.0, iou)

        max_p = jnp.max(iou_c, axis=1, keepdims=True)
        max_t = jnp.max(iou_r, axis=0, keepdims=True)

        rid = jax.lax.broadcasted_iota(jnp.int32, (K, 1), 0)
        cid = jax.lax.broadcasted_iota(jnp.int32, (1, K), 1)
        pres_p = (area_p > 0) & (rid > 0)
        pres_t = (area_t > 0) & (cid > 0)

        loss = (jnp.sum(jnp.where(pres_p, 1.0 - max_p, 0.0),
                        axis=0, keepdims=True)
                + jnp.sum(jnp.where(pres_t, 1.0 - max_t, 0.0),
                          axis=1, keepdims=True))
        n = (jnp.sum(pres_p.astype(jnp.float32), axis=0, keepdims=True)
             + jnp.sum(pres_t.astype(jnp.float32), axis=1, keepdims=True))
        out_ref[...] = jnp.where(n > 0, loss / jnp.maximum(n, 1.0), 0.0)


def kernel(pred_mask, true_mask):
    loss = pl.pallas_call(
        _body,
        out_shape=jax.ShapeDtypeStruct((1, 1), jnp.float32),
        grid=(NS,),
        in_specs=[
            pl.BlockSpec((1, R, W), lambda s: (0, s, 0)),
            pl.BlockSpec((1, R, W), lambda s: (0, s, 0)),
        ],
        out_specs=pl.BlockSpec((1, 1), lambda s: (0, 0)),
        scratch_shapes=[pltpu.VMEM((K, K), jnp.float32)],
        name="iou_loss_fused",
    )(pred_mask, true_mask)
    return loss[0, 0]


# speedup vs baseline: 2.0689x; 1.0751x over previous
import jax
import jax.numpy as jnp
from jax.experimental import pallas as pl
from jax.experimental.pallas import tpu as pltpu

K = 64
H = 1024
W = 1024
R = 64
NS = H // R


def _body(pm_ref, tm_ref, out_ref, acc_ref):
    s = pl.program_id(0)
    pm = pm_ref[0].astype(jnp.bfloat16)
    tm = tm_ref[0].astype(jnp.bfloat16)
    ids = jax.lax.broadcasted_iota(jnp.int32, (K, 1), 0).astype(jnp.bfloat16)
    one = jnp.bfloat16(1.0)
    zero = jnp.bfloat16(0.0)

    p_oh = jnp.concatenate(
        [jnp.where(pm[r:r + 1, :] == ids, one, zero) for r in range(R)],
        axis=1)
    t_oh = jnp.concatenate(
        [jnp.where(tm[r:r + 1, :] == ids, one, zero) for r in range(R)],
        axis=1)
    part = jax.lax.dot_general(
        p_oh, t_oh, (((1,), (1,)), ((), ())),
        preferred_element_type=jnp.float32)

    @pl.when(s == 0)
    def _init():
        acc_ref[...] = part

    @pl.when(s > 0)
    def _acc():
        acc_ref[...] += part

    @pl.when(s == NS - 1)
    def _finalize():
        inter = acc_ref[...]
        area_p = jnp.sum(inter, axis=1, keepdims=True)
        area_t = jnp.sum(inter, axis=0, keepdims=True)
        union = area_p + area_t - inter
        iou = jnp.where(union > 0, inter / jnp.maximum(union, 1.0), 0.0)

        col = jax.lax.broadcasted_iota(jnp.int32, (K, K), 1)
        row = jax.lax.broadcasted_iota(jnp.int32, (K, K), 0)
        iou_c = jnp.where(col == 0, 0.0, iou)
        iou_r = jnp.where(row == 0, 0.0, iou)

        max_p = jnp.max(iou_c, axis=1, keepdims=True)
        max_t = jnp.max(iou_r, axis=0, keepdims=True)

        rid = jax.lax.broadcasted_iota(jnp.int32, (K, 1), 0)
        cid = jax.lax.broadcasted_iota(jnp.int32, (1, K), 1)
        pres_p = (area_p > 0) & (rid > 0)
        pres_t = (area_t > 0) & (cid > 0)

        loss = (jnp.sum(jnp.where(pres_p, 1.0 - max_p, 0.0),
                        axis=0, keepdims=True)
                + jnp.sum(jnp.where(pres_t, 1.0 - max_t, 0.0),
                          axis=1, keepdims=True))
        n = (jnp.sum(pres_p.astype(jnp.float32), axis=0, keepdims=True)
             + jnp.sum(pres_t.astype(jnp.float32), axis=1, keepdims=True))
        out_ref[...] = jnp.where(n > 0, loss / jnp.maximum(n, 1.0), 0.0)


def kernel(pred_mask, true_mask):
    loss = pl.pallas_call(
        _body,
        out_shape=jax.ShapeDtypeStruct((1, 1), jnp.float32),
        grid=(NS,),
        in_specs=[
            pl.BlockSpec((1, R, W), lambda s: (0, s, 0)),
            pl.BlockSpec((1, R, W), lambda s: (0, s, 0)),
        ],
        out_specs=pl.BlockSpec((1, 1), lambda s: (0, 0)),
        scratch_shapes=[pltpu.VMEM((K, K), jnp.float32)],
        name="iou_loss_fused",
    )(pred_mask, true_mask)
    return loss[0, 0]


# speedup vs baseline: 2.1383x; 1.0336x over previous
import jax
import jax.numpy as jnp
from jax.experimental import pallas as pl
from jax.experimental.pallas import tpu as pltpu

K = 64
H = 1024
W = 1024
R = 128
NS = H // R


def _body(pm_ref, tm_ref, out_ref, acc_ref):
    s = pl.program_id(0)
    pm = pm_ref[0].astype(jnp.bfloat16)
    tm = tm_ref[0].astype(jnp.bfloat16)
    ids = jax.lax.broadcasted_iota(jnp.int32, (K, 1), 0).astype(jnp.bfloat16)
    one = jnp.bfloat16(1.0)
    zero = jnp.bfloat16(0.0)

    p_oh = jnp.concatenate(
        [jnp.where(pm[r:r + 1, :] == ids, one, zero) for r in range(R)],
        axis=1)
    t_oh = jnp.concatenate(
        [jnp.where(tm[r:r + 1, :] == ids, one, zero) for r in range(R)],
        axis=1)
    part = jax.lax.dot_general(
        p_oh, t_oh, (((1,), (1,)), ((), ())),
        preferred_element_type=jnp.float32)

    @pl.when(s == 0)
    def _init():
        acc_ref[...] = part

    @pl.when(s > 0)
    def _acc():
        acc_ref[...] += part

    @pl.when(s == NS - 1)
    def _finalize():
        inter = acc_ref[...]
        area_p = jnp.sum(inter, axis=1, keepdims=True)
        area_t = jnp.sum(inter, axis=0, keepdims=True)
        union = area_p + area_t - inter
        iou = jnp.where(union > 0, inter / jnp.maximum(union, 1.0), 0.0)

        col = jax.lax.broadcasted_iota(jnp.int32, (K, K), 1)
        row = jax.lax.broadcasted_iota(jnp.int32, (K, K), 0)
        iou_c = jnp.where(col == 0, 0.0, iou)
        iou_r = jnp.where(row == 0, 0.0, iou)

        max_p = jnp.max(iou_c, axis=1, keepdims=True)
        max_t = jnp.max(iou_r, axis=0, keepdims=True)

        rid = jax.lax.broadcasted_iota(jnp.int32, (K, 1), 0)
        cid = jax.lax.broadcasted_iota(jnp.int32, (1, K), 1)
        pres_p = (area_p > 0) & (rid > 0)
        pres_t = (area_t > 0) & (cid > 0)

        loss = (jnp.sum(jnp.where(pres_p, 1.0 - max_p, 0.0),
                        axis=0, keepdims=True)
                + jnp.sum(jnp.where(pres_t, 1.0 - max_t, 0.0),
                          axis=1, keepdims=True))
        n = (jnp.sum(pres_p.astype(jnp.float32), axis=0, keepdims=True)
             + jnp.sum(pres_t.astype(jnp.float32), axis=1, keepdims=True))
        out_ref[...] = jnp.where(n > 0, loss / jnp.maximum(n, 1.0), 0.0)


def kernel(pred_mask, true_mask):
    loss = pl.pallas_call(
        _body,
        out_shape=jax.ShapeDtypeStruct((1, 1), jnp.float32),
        grid=(NS,),
        in_specs=[
            pl.BlockSpec((1, R, W), lambda s: (0, s, 0)),
            pl.BlockSpec((1, R, W), lambda s: (0, s, 0)),
        ],
        out_specs=pl.BlockSpec((1, 1), lambda s: (0, 0)),
        scratch_shapes=[pltpu.VMEM((K, K), jnp.float32)],
        name="iou_loss_fused",
    )(pred_mask, true_mask)
    return loss[0, 0]


# speedup vs baseline: 2.9261x; 1.3684x over previous
import jax
import jax.numpy as jnp
from jax.experimental import pallas as pl
from jax.experimental.pallas import tpu as pltpu

K = 64
H = 1024
W = 1024
R = 128
NS = H // R


def _body(pm_ref, tm_ref, out_ref, acc_ref):
    s = pl.program_id(0)
    pm = pm_ref[0].astype(jnp.bfloat16)
    tm = tm_ref[0].astype(jnp.bfloat16)
    ids = jax.lax.broadcasted_iota(jnp.int32, (K, 1), 0).astype(jnp.bfloat16)
    one = jnp.bfloat16(1.0)
    zero = jnp.bfloat16(0.0)

    f8 = jnp.float8_e4m3fn
    p_oh = jnp.concatenate(
        [jnp.where(pm[r:r + 1, :] == ids, one, zero) for r in range(R)],
        axis=1).astype(f8)
    t_oh = jnp.concatenate(
        [jnp.where(tm[r:r + 1, :] == ids, one, zero) for r in range(R)],
        axis=1).astype(f8)
    part = jax.lax.dot_general(
        p_oh, t_oh, (((1,), (1,)), ((), ())),
        preferred_element_type=jnp.float32)

    @pl.when(s == 0)
    def _init():
        acc_ref[...] = part

    @pl.when(s > 0)
    def _acc():
        acc_ref[...] += part

    @pl.when(s == NS - 1)
    def _finalize():
        inter = acc_ref[...]
        area_p = jnp.sum(inter, axis=1, keepdims=True)
        area_t = jnp.sum(inter, axis=0, keepdims=True)
        union = area_p + area_t - inter
        iou = jnp.where(union > 0, inter / jnp.maximum(union, 1.0), 0.0)

        col = jax.lax.broadcasted_iota(jnp.int32, (K, K), 1)
        row = jax.lax.broadcasted_iota(jnp.int32, (K, K), 0)
        iou_c = jnp.where(col == 0, 0.0, iou)
        iou_r = jnp.where(row == 0, 0.0, iou)

        max_p = jnp.max(iou_c, axis=1, keepdims=True)
        max_t = jnp.max(iou_r, axis=0, keepdims=True)

        rid = jax.lax.broadcasted_iota(jnp.int32, (K, 1), 0)
        cid = jax.lax.broadcasted_iota(jnp.int32, (1, K), 1)
        pres_p = (area_p > 0) & (rid > 0)
        pres_t = (area_t > 0) & (cid > 0)

        loss = (jnp.sum(jnp.where(pres_p, 1.0 - max_p, 0.0),
                        axis=0, keepdims=True)
                + jnp.sum(jnp.where(pres_t, 1.0 - max_t, 0.0),
                          axis=1, keepdims=True))
        n = (jnp.sum(pres_p.astype(jnp.float32), axis=0, keepdims=True)
             + jnp.sum(pres_t.astype(jnp.float32), axis=1, keepdims=True))
        out_ref[...] = jnp.where(n > 0, loss / jnp.maximum(n, 1.0), 0.0)


def kernel(pred_mask, true_mask):
    loss = pl.pallas_call(
        _body,
        out_shape=jax.ShapeDtypeStruct((1, 1), jnp.float32),
        grid=(NS,),
        in_specs=[
            pl.BlockSpec((1, R, W), lambda s: (0, s, 0)),
            pl.BlockSpec((1, R, W), lambda s: (0, s, 0)),
        ],
        out_specs=pl.BlockSpec((1, 1), lambda s: (0, 0)),
        scratch_shapes=[pltpu.VMEM((K, K), jnp.float32)],
        name="iou_loss_fused",
    )(pred_mask, true_mask)
    return loss[0, 0]
